# baseline (speedup 1.0000x reference)
import functools

import jax
import jax.numpy as jnp
import numpy as np
from jax import lax
from jax.experimental import pallas as pl
from jax.experimental.pallas import tpu as pltpu
from jax.experimental.pallas import tpu_sc as plsc

_C = 19
_T = 2048
_NW = 32
_IGNORE = 255
_LN2 = 0.6931471805599453
_SQRT2 = 1.4142135623730951


def _ln16(x):
    bits = lax.bitcast_convert_type(x, jnp.int32)
    e = ((bits >> 23) & 0xFF) - 127
    m = lax.bitcast_convert_type(
        (bits & 0x007FFFFF) | 0x3F800000, jnp.float32)
    big = m > _SQRT2
    m = jnp.where(big, m * 0.5, m)
    ef = jnp.where(big, e + 1, e).astype(jnp.float32)
    z = (m - 1.0) / (m + 1.0)
    z2 = z * z
    p = jnp.float32(1.0 / 9.0)
    p = p * z2 + jnp.float32(1.0 / 7.0)
    p = p * z2 + jnp.float32(1.0 / 5.0)
    p = p * z2 + jnp.float32(1.0 / 3.0)
    p = p * z2 + 1.0
    return ef * jnp.float32(_LN2) + (2.0 * z) * p


def _sc_body(logits_hbm, labels_hbm, parts_hbm, loss_hbm, xbuf, lbuf, obuf,
             stage, cnt_acc, sm_acc, vd_acc, *, n_pix, thresh):
    nc = 2
    wid = lax.axis_index("s") * nc + lax.axis_index("c")
    pw = n_pix // _NW
    per_b = logits_hbm.shape[2]
    wpb = per_b // pw
    b = wid // wpb
    base = (wid % wpb) * pw

    def g_body(g, carry):
        ds = pl.ds(g * 16, 16)
        lab = lbuf[ds]
        v = xbuf[0, ds]
        acc_e = jnp.exp(v)
        acc_l = jnp.where(lab == 0, v, jnp.zeros_like(v))
        for c in range(1, _C):
            v = xbuf[c, ds]
            acc_e = acc_e + jnp.exp(v)
            acc_l = jnp.where(lab == c, v, acc_l)
        zf16 = jnp.zeros_like(acc_e)
        of16 = zf16 + 1.0
        valid = lab != _IGNORE
        loss = jnp.where(valid, _ln16(acc_e) - acc_l, zf16)
        obuf[ds] = loss
        mask = loss > thresh
        cnt_acc[...] = cnt_acc[...] + jnp.where(mask, of16, zf16)
        sm_acc[...] = sm_acc[...] + jnp.where(mask, loss, zf16)
        vd_acc[...] = vd_acc[...] + jnp.where(valid, of16, zf16)
        return carry

    def tile_body(i, carry):
        off = base + i * _T
        pltpu.sync_copy(logits_hbm.at[b, :, pl.ds(off, _T)], xbuf)
        pltpu.sync_copy(labels_hbm.at[b, pl.ds(off, _T)], lbuf)
        carry = lax.fori_loop(0, _T // 16, g_body, carry)
        pltpu.sync_copy(obuf, loss_hbm.at[b, pl.ds(off, _T)])
        return carry

    zf = jnp.zeros((16,), jnp.float32)
    cnt_acc[...] = zf
    sm_acc[...] = zf
    vd_acc[...] = zf
    lax.fori_loop(0, pw // _T, tile_body, 0)

    stage[pl.ds(0, 16)] = cnt_acc[...]
    stage[pl.ds(16, 16)] = sm_acc[...]
    stage[pl.ds(32, 16)] = vd_acc[...]
    stage[pl.ds(48, 16)] = zf
    pltpu.sync_copy(stage, parts_hbm.at[wid])


def _fin_body(parts_ref, loss_hbm, out_ref, lscr, sem, *, n_min_static,
              shape2d):
    parts = parts_ref[...]
    n_hard_f = jnp.sum(parts[:, 0:16])
    hard_sum = jnp.sum(parts[:, 16:32])
    n_valid_f = jnp.sum(parts[:, 32:48])
    n_hard = n_hard_f.astype(jnp.int32)
    n_min = n_valid_f.astype(jnp.int32) // 16
    few = n_hard < n_min

    @pl.when(jnp.logical_not(few))
    def _many():
        out_ref[0] = hard_sum / n_hard_f

    @pl.when(few)
    def _few():
        cp = pltpu.make_async_copy(loss_hbm, lscr, sem)
        cp.start()
        cp.wait()
        k = n_min_static
        lv = lscr[...]
        bits = lax.bitcast_convert_type(lv, jnp.int32)

        def body(j, ans):
            trial = ans | (1 << (30 - j))
            c = jnp.sum((bits > trial).astype(jnp.int32))
            return jnp.where(c >= k, trial, ans)

        ans = lax.fori_loop(0, 31, body, jnp.int32(0))
        c0 = jnp.sum((bits > 0).astype(jnp.int32))
        tbits = jnp.where(c0 >= k, ans + 1, 0)
        t = lax.bitcast_convert_type(tbits, jnp.float32)
        gt = bits > tbits
        cnt_gt = jnp.sum(gt.astype(jnp.int32))
        sum_gt = jnp.sum(jnp.where(gt, lv, 0.0))
        out_ref[0] = (
            sum_gt + (k - cnt_gt).astype(jnp.float32) * t
        ) / jnp.float32(k)


def kernel(logits, labels):
    b, c, h, w = logits.shape
    per_b = h * w
    n_pix = b * per_b
    thresh = float(-np.log(np.float32(0.7)))

    logits3 = logits.reshape(b, c, per_b)
    labels2 = labels.reshape(b, per_b)

    mesh = plsc.VectorSubcoreMesh(core_axis_name="c", subcore_axis_name="s")
    sc = pl.kernel(
        functools.partial(_sc_body, n_pix=n_pix, thresh=thresh),
        out_type=(
            jax.ShapeDtypeStruct((_NW, 64), jnp.float32),
            jax.ShapeDtypeStruct((b, per_b), jnp.float32),
        ),
        mesh=mesh,
        scratch_types=[
            pltpu.VMEM((_C, _T), jnp.float32),
            pltpu.VMEM((_T,), jnp.int32),
            pltpu.VMEM((_T,), jnp.float32),
            pltpu.VMEM((64,), jnp.float32),
            pltpu.VMEM((16,), jnp.float32),
            pltpu.VMEM((16,), jnp.float32),
            pltpu.VMEM((16,), jnp.float32),
        ],
    )
    parts, loss = sc(logits3, labels2)

    fin = pl.pallas_call(
        functools.partial(_fin_body, n_min_static=n_pix // 16,
                          shape2d=(b, per_b)),
        in_specs=[
            pl.BlockSpec((_NW, 64), lambda: (0, 0)),
            pl.BlockSpec(memory_space=pl.ANY),
        ],
        out_specs=pl.BlockSpec(memory_space=pltpu.SMEM),
        out_shape=jax.ShapeDtypeStruct((1,), jnp.float32),
        scratch_shapes=[
            pltpu.VMEM((b, per_b), jnp.float32),
            pltpu.SemaphoreType.DMA,
        ],
    )(parts, loss)
    return fin[0]


# speedup vs baseline: 1.0792x; 1.0792x over previous
import functools

import jax
import jax.numpy as jnp
import numpy as np
from jax import lax
from jax.experimental import pallas as pl
from jax.experimental.pallas import tpu as pltpu
from jax.experimental.pallas import tpu_sc as plsc

_C = 19
_T = 2048
_NW = 32
_IGNORE = 255
_LN2 = 0.6931471805599453
_SQRT2 = 1.4142135623730951


def _ln16(x):
    bits = lax.bitcast_convert_type(x, jnp.int32)
    e = ((bits >> 23) & 0xFF) - 127
    m = lax.bitcast_convert_type(
        (bits & 0x007FFFFF) | 0x3F800000, jnp.float32)
    big = m > _SQRT2
    m = jnp.where(big, m * 0.5, m)
    ef = jnp.where(big, e + 1, e).astype(jnp.float32)
    z = (m - 1.0) / (m + 1.0)
    z2 = z * z
    p = jnp.float32(1.0 / 9.0)
    p = p * z2 + jnp.float32(1.0 / 7.0)
    p = p * z2 + jnp.float32(1.0 / 5.0)
    p = p * z2 + jnp.float32(1.0 / 3.0)
    p = p * z2 + 1.0
    return ef * jnp.float32(_LN2) + (2.0 * z) * p


def _sc_body(logits_hbm, labels_hbm, parts_hbm, loss_hbm, xbuf, lbuf, obuf,
             stage, cnt_acc, sm_acc, vd_acc, *, n_pix, thresh):
    nc = 2
    wid = lax.axis_index("s") * nc + lax.axis_index("c")
    pw = n_pix // _NW
    per_b = logits_hbm.shape[2]
    wpb = per_b // pw
    b = wid // wpb
    base = (wid % wpb) * pw

    def _tree(vals):
        while len(vals) > 1:
            vals = [a + b for a, b in zip(vals[::2], vals[1::2])] + (
                [vals[-1]] if len(vals) % 2 else [])
        return vals[0]

    def g_body(g, carry):
        cnt = cnt_acc[...]
        sm = sm_acc[...]
        vd = vd_acc[...]
        for half in range(2):
            ds = pl.ds(g * 32 + half * 16, 16)
            lab = lbuf[ds]
            vs = [xbuf[c, ds] for c in range(_C)]
            zf16 = jnp.zeros_like(vs[0])
            of16 = zf16 + 1.0
            acc_e = _tree([jnp.exp(v) for v in vs])
            acc_l = _tree(
                [jnp.where(lab == c, vs[c], zf16) for c in range(_C)])
            valid = lab != _IGNORE
            loss = jnp.where(valid, _ln16(acc_e) - acc_l, zf16)
            obuf[ds] = loss
            mask = loss > thresh
            cnt = cnt + jnp.where(mask, of16, zf16)
            sm = sm + jnp.where(mask, loss, zf16)
            vd = vd + jnp.where(valid, of16, zf16)
        cnt_acc[...] = cnt
        sm_acc[...] = sm
        vd_acc[...] = vd
        return carry

    def tile_body(i, carry):
        off = base + i * _T
        pltpu.sync_copy(logits_hbm.at[b, :, pl.ds(off, _T)], xbuf)
        pltpu.sync_copy(labels_hbm.at[b, pl.ds(off, _T)], lbuf)
        carry = lax.fori_loop(0, _T // 32, g_body, carry)
        pltpu.sync_copy(obuf, loss_hbm.at[b, pl.ds(off, _T)])
        return carry

    zf = jnp.zeros((16,), jnp.float32)
    cnt_acc[...] = zf
    sm_acc[...] = zf
    vd_acc[...] = zf
    lax.fori_loop(0, pw // _T, tile_body, 0)

    stage[pl.ds(0, 16)] = cnt_acc[...]
    stage[pl.ds(16, 16)] = sm_acc[...]
    stage[pl.ds(32, 16)] = vd_acc[...]
    stage[pl.ds(48, 16)] = zf
    pltpu.sync_copy(stage, parts_hbm.at[wid])


def _fin_body(parts_ref, loss_hbm, out_ref, lscr, sem, *, n_min_static,
              shape2d):
    parts = parts_ref[...]
    n_hard_f = jnp.sum(parts[:, 0:16])
    hard_sum = jnp.sum(parts[:, 16:32])
    n_valid_f = jnp.sum(parts[:, 32:48])
    n_hard = n_hard_f.astype(jnp.int32)
    n_min = n_valid_f.astype(jnp.int32) // 16
    few = n_hard < n_min

    @pl.when(jnp.logical_not(few))
    def _many():
        out_ref[0] = hard_sum / n_hard_f

    @pl.when(few)
    def _few():
        cp = pltpu.make_async_copy(loss_hbm, lscr, sem)
        cp.start()
        cp.wait()
        k = n_min_static
        lv = lscr[...]
        bits = lax.bitcast_convert_type(lv, jnp.int32)

        def body(j, ans):
            trial = ans | (1 << (30 - j))
            c = jnp.sum((bits > trial).astype(jnp.int32))
            return jnp.where(c >= k, trial, ans)

        ans = lax.fori_loop(0, 31, body, jnp.int32(0))
        c0 = jnp.sum((bits > 0).astype(jnp.int32))
        tbits = jnp.where(c0 >= k, ans + 1, 0)
        t = lax.bitcast_convert_type(tbits, jnp.float32)
        gt = bits > tbits
        cnt_gt = jnp.sum(gt.astype(jnp.int32))
        sum_gt = jnp.sum(jnp.where(gt, lv, 0.0))
        out_ref[0] = (
            sum_gt + (k - cnt_gt).astype(jnp.float32) * t
        ) / jnp.float32(k)


def kernel(logits, labels):
    b, c, h, w = logits.shape
    per_b = h * w
    n_pix = b * per_b
    thresh = float(-np.log(np.float32(0.7)))

    logits3 = logits.reshape(b, c, per_b)
    labels2 = labels.reshape(b, per_b)

    mesh = plsc.VectorSubcoreMesh(core_axis_name="c", subcore_axis_name="s")
    sc = pl.kernel(
        functools.partial(_sc_body, n_pix=n_pix, thresh=thresh),
        out_type=(
            jax.ShapeDtypeStruct((_NW, 64), jnp.float32),
            jax.ShapeDtypeStruct((b, per_b), jnp.float32),
        ),
        mesh=mesh,
        scratch_types=[
            pltpu.VMEM((_C, _T), jnp.float32),
            pltpu.VMEM((_T,), jnp.int32),
            pltpu.VMEM((_T,), jnp.float32),
            pltpu.VMEM((64,), jnp.float32),
            pltpu.VMEM((16,), jnp.float32),
            pltpu.VMEM((16,), jnp.float32),
            pltpu.VMEM((16,), jnp.float32),
        ],
    )
    parts, loss = sc(logits3, labels2)

    fin = pl.pallas_call(
        functools.partial(_fin_body, n_min_static=n_pix // 16,
                          shape2d=(b, per_b)),
        in_specs=[
            pl.BlockSpec((_NW, 64), lambda: (0, 0)),
            pl.BlockSpec(memory_space=pl.ANY),
        ],
        out_specs=pl.BlockSpec(memory_space=pltpu.SMEM),
        out_shape=jax.ShapeDtypeStruct((1,), jnp.float32),
        scratch_shapes=[
            pltpu.VMEM((b, per_b), jnp.float32),
            pltpu.SemaphoreType.DMA,
        ],
    )(parts, loss)
    return fin[0]


# speedup vs baseline: 1.3052x; 1.2094x over previous
import functools

import jax
import jax.numpy as jnp
import numpy as np
from jax import lax
from jax.experimental import pallas as pl
from jax.experimental.pallas import tpu as pltpu
from jax.experimental.pallas import tpu_sc as plsc

_C = 19
_SUB = 128
_HALF = _SUB // 2
_LANE = 1024
_CHUNK = _SUB * _LANE
_T = 2048
_NW = 32
_SC_B = 2
_IGNORE = 255
_LN2 = 0.6931471805599453
_SQRT2 = 1.4142135623730951


def _ln16(x):
    bits = lax.bitcast_convert_type(x, jnp.int32)
    e = ((bits >> 23) & 0xFF) - 127
    m = lax.bitcast_convert_type(
        (bits & 0x007FFFFF) | 0x3F800000, jnp.float32)
    big = m > _SQRT2
    m = jnp.where(big, m * 0.5, m)
    ef = jnp.where(big, e + 1, e).astype(jnp.float32)
    z = (m - 1.0) / (m + 1.0)
    z2 = z * z
    p = jnp.float32(1.0 / 9.0)
    p = p * z2 + jnp.float32(1.0 / 7.0)
    p = p * z2 + jnp.float32(1.0 / 5.0)
    p = p * z2 + jnp.float32(1.0 / 3.0)
    p = p * z2 + 1.0
    return ef * jnp.float32(_LN2) + (2.0 * z) * p


def _sc_body(logits_hbm, labels_hbm, parts_hbm, loss_hbm, xbuf, lbuf, obuf,
             stage, cnt_acc, sm_acc, vd_acc, *, b0, thresh):
    nc = 2
    wid = lax.axis_index("s") * nc + lax.axis_index("c")
    per_b = logits_hbm.shape[2]
    n_sc_pix = _SC_B * per_b
    pw = n_sc_pix // _NW
    wpb = per_b // pw
    b = b0 + wid // wpb
    base = (wid % wpb) * pw

    def _tree(vals):
        while len(vals) > 1:
            vals = [a + b_ for a, b_ in zip(vals[::2], vals[1::2])] + (
                [vals[-1]] if len(vals) % 2 else [])
        return vals[0]

    def g_body(g, carry):
        cnt = cnt_acc[...]
        sm = sm_acc[...]
        vd = vd_acc[...]
        for half in range(2):
            ds = pl.ds(g * 32 + half * 16, 16)
            lab = lbuf[ds]
            vs = [xbuf[c, ds] for c in range(_C)]
            zf16 = jnp.zeros_like(vs[0])
            of16 = zf16 + 1.0
            acc_e = _tree([jnp.exp(v) for v in vs])
            acc_l = _tree(
                [jnp.where(lab == c, vs[c], zf16) for c in range(_C)])
            valid = lab != _IGNORE
            loss = jnp.where(valid, _ln16(acc_e) - acc_l, zf16)
            obuf[ds] = loss
            mask = loss > thresh
            cnt = cnt + jnp.where(mask, of16, zf16)
            sm = sm + jnp.where(mask, loss, zf16)
            vd = vd + jnp.where(valid, of16, zf16)
        cnt_acc[...] = cnt
        sm_acc[...] = sm
        vd_acc[...] = vd
        return carry

    def tile_body(i, carry):
        off = base + i * _T
        pltpu.sync_copy(logits_hbm.at[b, :, pl.ds(off, _T)], xbuf)
        pltpu.sync_copy(labels_hbm.at[b, pl.ds(off, _T)], lbuf)
        carry = lax.fori_loop(0, _T // 32, g_body, carry)
        pltpu.sync_copy(obuf, loss_hbm.at[b - b0, pl.ds(off, _T)])
        return carry

    zf = jnp.zeros((16,), jnp.float32)
    cnt_acc[...] = zf
    sm_acc[...] = zf
    vd_acc[...] = zf
    lax.fori_loop(0, pw // _T, tile_body, 0)

    stage[pl.ds(0, 16)] = cnt_acc[...]
    stage[pl.ds(16, 16)] = sm_acc[...]
    stage[pl.ds(32, 16)] = vd_acc[...]
    stage[pl.ds(48, 16)] = zf
    pltpu.sync_copy(stage, parts_hbm.at[wid])


def _tc_body(logits_a, logits_b, labels_ref, loss_out, sums_ref, cnt_acc,
             sum_acc, vld_acc, *, n_steps, thresh):
    i = pl.program_id(0)

    lab = labels_ref[0]

    @pl.when(i == 0)
    def _init():
        cnt_acc[...] = jnp.zeros_like(cnt_acc)
        sum_acc[...] = jnp.zeros_like(sum_acc)
        vld_acc[...] = jnp.zeros_like(vld_acc)

    for h, ref in ((0, logits_a), (1, logits_b)):
        labh = lab[h * _HALF:(h + 1) * _HALF]
        acc_e = jnp.zeros((_HALF, _LANE), jnp.float32)
        acc_l = jnp.zeros((_HALF, _LANE), jnp.float32)
        for c in range(_C):
            s = ref[0, c, 0, 0]
            acc_e += jnp.exp(s)
            acc_l = jnp.where(labh == c, s, acc_l)

        valid = labh != _IGNORE
        loss = jnp.where(valid, jnp.log(acc_e) - acc_l, 0.0)
        loss_out[0, h * _HALF:(h + 1) * _HALF] = loss
        mask = loss > thresh
        cnt_acc[...] += mask.astype(jnp.float32)
        sum_acc[...] += jnp.where(mask, loss, 0.0)
        vld_acc[...] += valid.astype(jnp.float32)

    @pl.when(i == n_steps - 1)
    def _finalize():
        sums_ref[0] = jnp.sum(cnt_acc[...])
        sums_ref[1] = jnp.sum(sum_acc[...])
        sums_ref[2] = jnp.sum(vld_acc[...])


def _fin_body(parts_ref, tcsums_ref, loss_tc, loss_sc, out_ref, lscr, sem,
              *, n_min_static, tc_rows):
    parts = parts_ref[...]
    n_hard_f = jnp.sum(parts[:, 0:16]) + tcsums_ref[0]
    hard_sum = jnp.sum(parts[:, 16:32]) + tcsums_ref[1]
    n_valid_f = jnp.sum(parts[:, 32:48]) + tcsums_ref[2]
    n_hard = n_hard_f.astype(jnp.int32)
    n_min = n_valid_f.astype(jnp.int32) // 16
    few = n_hard < n_min

    @pl.when(jnp.logical_not(few))
    def _many():
        out_ref[0] = hard_sum / n_hard_f

    @pl.when(few)
    def _few():
        cp1 = pltpu.make_async_copy(loss_tc, lscr.at[pl.ds(0, tc_rows)], sem)
        cp1.start()
        cp1.wait()
        cp2 = pltpu.make_async_copy(
            loss_sc, lscr.at[pl.ds(tc_rows, lscr.shape[0] - tc_rows)], sem)
        cp2.start()
        cp2.wait()
        k = n_min_static
        lv = lscr[...]
        bits = lax.bitcast_convert_type(lv, jnp.int32)

        def body(j, ans):
            trial = ans | (1 << (30 - j))
            c = jnp.sum((bits > trial).astype(jnp.int32))
            return jnp.where(c >= k, trial, ans)

        ans = lax.fori_loop(0, 31, body, jnp.int32(0))
        c0 = jnp.sum((bits > 0).astype(jnp.int32))
        tbits = jnp.where(c0 >= k, ans + 1, 0)
        t = lax.bitcast_convert_type(tbits, jnp.float32)
        gt = bits > tbits
        cnt_gt = jnp.sum(gt.astype(jnp.int32))
        sum_gt = jnp.sum(jnp.where(gt, lv, 0.0))
        out_ref[0] = (
            sum_gt + (k - cnt_gt).astype(jnp.float32) * t
        ) / jnp.float32(k)


def kernel(logits, labels):
    b, c, h, w = logits.shape
    per_b = h * w
    n_pix = b * per_b
    tc_b = b - _SC_B
    chunks_per_b = per_b // _CHUNK
    n_steps = tc_b * chunks_per_b
    thresh = float(-np.log(np.float32(0.7)))

    logits3 = logits.reshape(b, c, per_b)
    labels2 = labels.reshape(b, per_b)

    mesh = plsc.VectorSubcoreMesh(core_axis_name="c", subcore_axis_name="s")
    sc = pl.kernel(
        functools.partial(_sc_body, b0=tc_b, thresh=thresh),
        out_type=(
            jax.ShapeDtypeStruct((_NW, 64), jnp.float32),
            jax.ShapeDtypeStruct((_SC_B, per_b), jnp.float32),
        ),
        mesh=mesh,
        scratch_types=[
            pltpu.VMEM((_C, _T), jnp.float32),
            pltpu.VMEM((_T,), jnp.int32),
            pltpu.VMEM((_T,), jnp.float32),
            pltpu.VMEM((64,), jnp.float32),
            pltpu.VMEM((16,), jnp.float32),
            pltpu.VMEM((16,), jnp.float32),
            pltpu.VMEM((16,), jnp.float32),
        ],
    )
    parts, loss_sc = sc(logits3, labels2)

    logits6 = logits.reshape(b, c, chunks_per_b, 2, _HALF, _LANE)
    labels3 = labels.reshape(b * chunks_per_b, _SUB, _LANE)

    spec_half = lambda half: pl.BlockSpec(
        (1, c, 1, 1, _HALF, _LANE),
        lambda i: (i // chunks_per_b, 0, i % chunks_per_b, half, 0, 0))

    loss_tc, tcsums = pl.pallas_call(
        functools.partial(_tc_body, n_steps=n_steps, thresh=thresh),
        grid=(n_steps,),
        in_specs=[
            spec_half(0),
            spec_half(1),
            pl.BlockSpec((1, _SUB, _LANE), lambda i: (i, 0, 0)),
        ],
        out_specs=[
            pl.BlockSpec((1, _SUB, _LANE), lambda i: (i, 0, 0)),
            pl.BlockSpec(memory_space=pltpu.SMEM),
        ],
        out_shape=[
            jax.ShapeDtypeStruct((n_steps, _SUB, _LANE), jnp.float32),
            jax.ShapeDtypeStruct((3,), jnp.float32),
        ],
        scratch_shapes=[
            pltpu.VMEM((_HALF, _LANE), jnp.float32),
            pltpu.VMEM((_HALF, _LANE), jnp.float32),
            pltpu.VMEM((_HALF, _LANE), jnp.float32),
        ],
    )(logits6, logits6, labels3)

    sc_rows = _SC_B * per_b // (_SUB * _LANE)
    loss_sc3 = loss_sc.reshape(sc_rows, _SUB, _LANE)

    fin = pl.pallas_call(
        functools.partial(_fin_body, n_min_static=n_pix // 16,
                          tc_rows=n_steps),
        in_specs=[
            pl.BlockSpec((_NW, 64), lambda: (0, 0)),
            pl.BlockSpec(memory_space=pltpu.SMEM),
            pl.BlockSpec(memory_space=pl.ANY),
            pl.BlockSpec(memory_space=pl.ANY),
        ],
        out_specs=pl.BlockSpec(memory_space=pltpu.SMEM),
        out_shape=jax.ShapeDtypeStruct((1,), jnp.float32),
        scratch_shapes=[
            pltpu.VMEM((n_steps + sc_rows, _SUB, _LANE), jnp.float32),
            pltpu.SemaphoreType.DMA,
        ],
    )(parts, tcsums, loss_tc, loss_sc3)
    return fin[0]


# speedup vs baseline: 2.5662x; 1.9662x over previous
import functools

import jax
import jax.numpy as jnp
import numpy as np
from jax.experimental import pallas as pl
from jax.experimental.pallas import tpu as pltpu

_C = 19
_SUB = 128
_HALF = _SUB // 2
_LANE = 1024
_CHUNK = _SUB * _LANE
_IGNORE = 255


def _ohem_kernel(logits_a, logits_b, labels_ref, out_ref, loss_scr, cnt_acc,
                 sum_acc, vld_acc, *, n_steps, n_min_static, thresh):
    i = pl.program_id(0)

    lab = labels_ref[0]

    @pl.when(i == 0)
    def _init():
        cnt_acc[...] = jnp.zeros_like(cnt_acc)
        sum_acc[...] = jnp.zeros_like(sum_acc)
        vld_acc[...] = jnp.zeros_like(vld_acc)

    for h, ref in ((0, logits_a), (1, logits_b)):
        labh = lab[h * _HALF:(h + 1) * _HALF]
        acc_e = jnp.zeros((_HALF, _LANE), jnp.float32)
        acc_l = jnp.zeros((_HALF, _LANE), jnp.float32)
        for c in range(_C):
            s = ref[0, c, 0, 0]
            acc_e += jnp.exp(s)
            acc_l = jnp.where(labh == c, s, acc_l)

        valid = labh != _IGNORE
        loss = jnp.where(valid, jnp.log(acc_e) - acc_l, 0.0)
        loss_scr[pl.ds(i, 1), h * _HALF:(h + 1) * _HALF] = loss[None]
        mask = loss > thresh
        cnt_acc[...] += mask.astype(jnp.int32)
        sum_acc[...] += jnp.where(mask, loss, 0.0)
        vld_acc[...] += valid.astype(jnp.int32)

    @pl.when(i == n_steps - 1)
    def _finalize():
        n_hard = jnp.sum(cnt_acc[...])
        hard_sum = jnp.sum(sum_acc[...])
        n_min = jnp.sum(vld_acc[...]) // 16
        few = n_hard < n_min

        @pl.when(jnp.logical_not(few))
        def _many():
            out_ref[0] = hard_sum / n_hard.astype(jnp.float32)

        @pl.when(few)
        def _few():
            k = n_min_static
            lv = loss_scr[...]
            bits = jax.lax.bitcast_convert_type(lv, jnp.int32)

            def body(j, ans):
                trial = ans | (1 << (30 - j))
                c = jnp.sum((bits > trial).astype(jnp.int32))
                return jnp.where(c >= k, trial, ans)

            ans = jax.lax.fori_loop(0, 31, body, jnp.int32(0))
            c0 = jnp.sum((bits > 0).astype(jnp.int32))
            tbits = jnp.where(c0 >= k, ans + 1, 0)
            t = jax.lax.bitcast_convert_type(tbits, jnp.float32)
            gt = bits > tbits
            cnt_gt = jnp.sum(gt.astype(jnp.int32))
            sum_gt = jnp.sum(jnp.where(gt, lv, 0.0))
            out_ref[0] = (
                sum_gt + (k - cnt_gt).astype(jnp.float32) * t
            ) / jnp.float32(k)


def kernel(logits, labels):
    b, c, h, w = logits.shape
    npix = b * h * w
    n_steps = npix // _CHUNK
    chunks_per_b = (h * w) // _CHUNK
    thresh = float(-np.log(np.float32(0.7)))

    logits6 = logits.reshape(b, c, chunks_per_b, 2, _HALF, _LANE)
    labels3 = labels.reshape(n_steps, _SUB, _LANE)

    body = functools.partial(
        _ohem_kernel,
        n_steps=n_steps,
        n_min_static=npix // 16,
        thresh=thresh,
    )

    spec_half = lambda half: pl.BlockSpec(
        (1, c, 1, 1, _HALF, _LANE),
        lambda i: (i // chunks_per_b, 0, i % chunks_per_b, half, 0, 0))

    out = pl.pallas_call(
        body,
        grid=(n_steps,),
        in_specs=[
            spec_half(0),
            spec_half(1),
            pl.BlockSpec((1, _SUB, _LANE), lambda i: (i, 0, 0)),
        ],
        out_specs=pl.BlockSpec(memory_space=pltpu.SMEM),
        out_shape=jax.ShapeDtypeStruct((1,), jnp.float32),
        scratch_shapes=[
            pltpu.VMEM((n_steps, _SUB, _LANE), jnp.float32),
            pltpu.VMEM((_HALF, _LANE), jnp.int32),
            pltpu.VMEM((_HALF, _LANE), jnp.float32),
            pltpu.VMEM((_HALF, _LANE), jnp.int32),
        ],
    )(logits6, logits6, labels3)
    return out[0]
